# Initial kernel scaffold; baseline (speedup 1.0000x reference)
import jax
import jax.numpy as jnp
from jax.experimental import pallas as pl


def kernel(x, table):
    raise NotImplementedError("write your pallas kernel here")


# speedup vs baseline: 1.4805x; 1.4805x over previous
import functools

import jax
import jax.numpy as jnp
from jax import lax
from jax.experimental import pallas as pl
from jax.experimental.pallas import tpu as pltpu
from jax.experimental.pallas import tpu_sc as plsc

_NC = 2
_NS = 16
_NW = _NC * _NS
_L = 16
_CHUNK = 64
_DP = 512
_MAXQ = 8


@functools.lru_cache(maxsize=None)
def _make_sc_lookup(N, V, D):
    bpw = N // _NW
    n_chunks = bpw // _CHUNK
    n_vregs_row = D // _L
    tail = D - n_vregs_row * _L
    mesh = plsc.VectorSubcoreMesh(core_axis_name="c", subcore_axis_name="s")

    @functools.partial(
        pl.kernel,
        mesh=mesh,
        out_type=jax.ShapeDtypeStruct((N * D,), jnp.float32),
        scratch_types=[
            pltpu.VMEM((bpw,), jnp.int32),
            pltpu.VMEM((_CHUNK, _DP), jnp.float32),
            pltpu.VMEM((1, _DP), jnp.float32),
            pltpu.VMEM((_CHUNK * D,), jnp.float32),
            pltpu.SemaphoreType.DMA,
        ],
        compiler_params=pltpu.CompilerParams(needs_layout_passes=False),
    )
    def body(x_hbm, table_hbm, out_hbm, idx_v, rows_v, row1_v, packed_v, sem):
        cid = lax.axis_index("c")
        sid = lax.axis_index("s")
        wid = sid * _NC + cid
        base = pl.multiple_of(wid * bpw, 8)

        pltpu.sync_copy(x_hbm.at[pl.ds(base, bpw)], idx_v)

        def fix(j, carry):
            v = idx_v[pl.ds(j * _L, _L)]
            v = jnp.where(v < V, v, V - 1)
            v = jnp.where(v < 0, v, 0)
            idx_v[pl.ds(j * _L, _L)] = v
            return carry

        lax.fori_loop(0, bpw // _L, fix, 0)

        def compact_row(r, c1):
            pbase = r * D

            def per_vreg(j, c2):
                packed_v[pl.ds(pbase + j * _L, _L)] = (
                    rows_v[r, pl.ds(j * _L, _L)])
                return c2

            lax.fori_loop(0, n_vregs_row, per_vreg, 0)
            if tail:
                packed_v[pl.ds(pbase + D - _L, _L)] = rows_v[r, pl.ds(D - _L, _L)]
            return c1

        def wait_one_write():
            pltpu.make_async_copy(
                packed_v, out_hbm.at[pl.ds(base * D, _CHUNK * D)], sem).wait()

        def chunk(i, carry):
            cached, outstanding = carry
            mx = jnp.int32(-(2 ** 31))
            mn = jnp.int32(2 ** 31 - 1)
            for j in range(_CHUNK // _L):
                v = idx_v[pl.ds(i * _CHUNK + j * _L, _L)]
                mx = jnp.maximum(mx, jnp.max(v))
                mn = jnp.minimum(mn, jnp.min(v))
            uniform = mx == mn
            hit = uniform & (mn == cached)

            @pl.when(jnp.logical_not(hit))
            def _miss():
                lax.fori_loop(0, outstanding,
                              lambda k, c: (wait_one_write(), c)[1], 0)

                @pl.when(uniform)
                def _one_row():
                    idx1 = idx_v.at[pl.ds(i * _CHUNK, 1)]
                    pltpu.async_copy(table_hbm.at[idx1], row1_v, sem).wait()

                    def first_row(j, c2):
                        packed_v[pl.ds(j * _L, _L)] = row1_v[0, pl.ds(j * _L, _L)]
                        return c2

                    lax.fori_loop(0, n_vregs_row, first_row, 0)
                    if tail:
                        packed_v[pl.ds(D - _L, _L)] = row1_v[0, pl.ds(D - _L, _L)]

                    def rep(r, c1):
                        def rep_vreg(j, c2):
                            packed_v[pl.ds(r * D + j * _L, _L)] = (
                                packed_v[pl.ds(j * _L, _L)])
                            return c2
                        lax.fori_loop(0, n_vregs_row, rep_vreg, 0)
                        if tail:
                            packed_v[pl.ds(r * D + D - _L, _L)] = (
                                packed_v[pl.ds(D - _L, _L)])
                        return c1

                    lax.fori_loop(1, _CHUNK, rep, 0)

                @pl.when(jnp.logical_not(uniform))
                def _general():
                    idx_slice = idx_v.at[pl.ds(i * _CHUNK, _CHUNK)]
                    pltpu.async_copy(table_hbm.at[idx_slice], rows_v, sem).wait()
                    lax.fori_loop(0, _CHUNK, compact_row, 0)

            outstanding2 = jnp.where(hit, outstanding, 0)

            @pl.when(outstanding2 >= _MAXQ)
            def _throttle():
                wait_one_write()

            outstanding3 = jnp.minimum(outstanding2, _MAXQ - 1)
            out_off = pl.multiple_of((base + i * _CHUNK) * D, 8)
            pltpu.async_copy(
                packed_v, out_hbm.at[pl.ds(out_off, _CHUNK * D)], sem)
            new_cached = jnp.where(uniform, mn, jnp.int32(-1))
            return (new_cached, outstanding3 + 1)

        _, outstanding_end = lax.fori_loop(
            0, n_chunks, chunk, (jnp.int32(-1), jnp.int32(0)))
        lax.fori_loop(0, outstanding_end,
                      lambda k, c: (wait_one_write(), c)[1], 0)

    return body


def kernel(x, table):
    B, F = x.shape
    V, D = table.shape
    xf = x.reshape(B * F)
    table_p = jnp.pad(table, ((0, 0), (0, _DP - D)))
    out = _make_sc_lookup(B * F, V, D)(xf, table_p)
    return out.reshape(B, F, D)


# speedup vs baseline: 2.8515x; 1.9260x over previous
import functools

import jax
import jax.numpy as jnp
from jax import lax
from jax.experimental import pallas as pl
from jax.experimental.pallas import tpu as pltpu
from jax.experimental.pallas import tpu_sc as plsc

_NC = 2
_NS = 16
_NW = _NC * _NS
_L = 16
_DP = 512
_BB = 128


@functools.lru_cache(maxsize=None)
def _make_sc_analyze(N, V):
    bpw = N // _NW
    mesh = plsc.VectorSubcoreMesh(core_axis_name="c", subcore_axis_name="s")

    @functools.partial(
        pl.kernel,
        mesh=mesh,
        out_type=(
            jax.ShapeDtypeStruct((N,), jnp.int32),
            jax.ShapeDtypeStruct((_NW * _L,), jnp.int32),
            jax.ShapeDtypeStruct((_NW, _DP), jnp.float32),
        ),
        scratch_types=[
            pltpu.VMEM((bpw,), jnp.int32),
            pltpu.VMEM((_NW * _L,), jnp.int32),
            pltpu.VMEM((1, _DP), jnp.float32),
            pltpu.SemaphoreType.DMA,
        ],
        compiler_params=pltpu.CompilerParams(needs_layout_passes=False),
    )
    def body(x_hbm, table_hbm, eff_hbm, flags_hbm, rows_hbm,
             idx_v, flag_v, row1_v, sem):
        cid = lax.axis_index("c")
        sid = lax.axis_index("s")
        wid = sid * _NC + cid
        base = pl.multiple_of(wid * bpw, 8)

        pltpu.sync_copy(x_hbm.at[pl.ds(base, bpw)], idx_v)

        def fix(j, carry):
            mn, mx = carry
            v = idx_v[pl.ds(j * _L, _L)]
            v = jnp.where(v < V, v, V - 1)
            v = jnp.where(v < 0, v, 0)
            idx_v[pl.ds(j * _L, _L)] = v
            return (jnp.minimum(mn, jnp.min(v)), jnp.maximum(mx, jnp.max(v)))

        mn, mx = lax.fori_loop(
            0, bpw // _L, fix,
            (jnp.int32(2 ** 31 - 1), jnp.int32(-(2 ** 31))))

        pltpu.sync_copy(idx_v, eff_hbm.at[pl.ds(base, bpw)])

        uniform = mn == mx
        fbase = pl.multiple_of(wid * _L, 8)
        flag_v[pl.ds(fbase, _L)] = jnp.where(
            uniform, jnp.int32(1), jnp.int32(0)) + jnp.zeros((_L,), jnp.int32)
        pltpu.sync_copy(flag_v.at[pl.ds(fbase, _L)],
                        flags_hbm.at[pl.ds(fbase, _L)])

        @pl.when(uniform)
        def _gather_row():
            idx1 = idx_v.at[pl.ds(0, 1)]
            pltpu.async_copy(table_hbm.at[idx1], row1_v, sem).wait()
            pltpu.sync_copy(row1_v, rows_hbm.at[pl.ds(wid, 1)])

    return body


@functools.lru_cache(maxsize=None)
def _make_tc_materialize(B, F, V, D):
    grid = B // _BB

    def body(flags_s, rows_ref, eff_ref, table_ref, out_ref):
        i = pl.program_id(0)
        flag = flags_s[i * _L]

        @pl.when(flag == 1)
        def _broadcast():
            row = rows_ref[pl.ds(i, 1), :D]
            out_ref[...] = jnp.broadcast_to(row[:, None, :], (_BB, F, D))

        @pl.when(flag != 1)
        def _general():
            def per_row(r, c):
                idx = eff_ref[pl.ds(r, 1), :]
                oh = (idx[0, :, None] ==
                      lax.broadcasted_iota(jnp.int32, (F, V), 1)
                      ).astype(jnp.float32)
                out_ref[pl.ds(r, 1)] = jnp.dot(
                    oh, table_ref[...],
                    preferred_element_type=jnp.float32)[None]
                return c

            lax.fori_loop(0, _BB, per_row, 0)

    grid_spec = pltpu.PrefetchScalarGridSpec(
        num_scalar_prefetch=1,
        grid=(grid,),
        in_specs=[
            pl.BlockSpec((_NW, _DP), lambda i, s: (0, 0)),
            pl.BlockSpec((_BB, F), lambda i, s: (i, 0)),
            pl.BlockSpec((V, D), lambda i, s: (0, 0)),
        ],
        out_specs=pl.BlockSpec((_BB, F, D), lambda i, s: (i, 0, 0)),
    )
    return pl.pallas_call(
        body,
        grid_spec=grid_spec,
        out_shape=jax.ShapeDtypeStruct((B, F, D), jnp.float32),
        compiler_params=pltpu.CompilerParams(
            dimension_semantics=("arbitrary",)),
    )


def kernel(x, table):
    B, F = x.shape
    V, D = table.shape
    xf = x.reshape(B * F)
    table_p = jnp.pad(table, ((0, 0), (0, _DP - D)))
    eff, flags, rows = _make_sc_analyze(B * F, V)(xf, table_p)
    eff2d = eff.reshape(B, F)
    return _make_tc_materialize(B, F, V, D)(flags, rows, eff2d, table)


# speedup vs baseline: 8.4301x; 2.9564x over previous
import functools

import jax
import jax.numpy as jnp
from jax import lax
from jax.experimental import pallas as pl
from jax.experimental.pallas import tpu as pltpu
from jax.experimental.pallas import tpu_sc as plsc

_NC = 2
_NS = 16
_NW = _NC * _NS
_L = 16
_BB = 128


@functools.lru_cache(maxsize=None)
def _make_sc_analyze(N, V):
    bpw = N // _NW
    mesh = plsc.VectorSubcoreMesh(core_axis_name="c", subcore_axis_name="s")

    @functools.partial(
        pl.kernel,
        mesh=mesh,
        out_type=(
            jax.ShapeDtypeStruct((N,), jnp.int32),
            jax.ShapeDtypeStruct((_NW * _L,), jnp.int32),
        ),
        scratch_types=[
            pltpu.VMEM((bpw,), jnp.int32),
            pltpu.VMEM((_NW * _L,), jnp.int32),
        ],
        compiler_params=pltpu.CompilerParams(needs_layout_passes=False),
    )
    def body(x_hbm, eff_hbm, flags_hbm, idx_v, flag_v):
        cid = lax.axis_index("c")
        sid = lax.axis_index("s")
        wid = sid * _NC + cid
        base = pl.multiple_of(wid * bpw, 8)

        pltpu.sync_copy(x_hbm.at[pl.ds(base, bpw)], idx_v)

        def fix(j, carry):
            mn, mx = carry
            v = idx_v[pl.ds(j * _L, _L)]
            v = jnp.where(v < V, v, V - 1)
            v = jnp.where(v < 0, v, 0)
            idx_v[pl.ds(j * _L, _L)] = v
            return (jnp.minimum(mn, jnp.min(v)), jnp.maximum(mx, jnp.max(v)))

        mn, mx = lax.fori_loop(
            0, bpw // _L, fix,
            (jnp.int32(2 ** 31 - 1), jnp.int32(-(2 ** 31))))

        pltpu.sync_copy(idx_v, eff_hbm.at[pl.ds(base, bpw)])

        flag = jnp.where(mn == mx, jnp.int32(1), jnp.int32(0))
        lane = lax.broadcasted_iota(jnp.int32, (_L,), 0)
        fbase = pl.multiple_of(wid * _L, 8)
        flag_v[pl.ds(fbase, _L)] = jnp.where(lane < 8, flag, mn)
        pltpu.sync_copy(flag_v.at[pl.ds(fbase, _L)],
                        flags_hbm.at[pl.ds(fbase, _L)])

    return body


@functools.lru_cache(maxsize=None)
def _make_tc_materialize(B, F, V, D):
    grid = B // _BB

    def body(flags_s, tablet_ref, efft_ref, out_ref):
        i = pl.program_id(0)
        flag = flags_s[i * _L]
        viota = lax.broadcasted_iota(jnp.int32, (V, _BB), 0)

        @pl.when(flag == 1)
        def _broadcast():
            u = flags_s[i * _L + 8]
            oh = (viota == u).astype(jnp.float32)
            col = jnp.dot(tablet_ref[...], oh,
                          precision=lax.Precision.HIGHEST,
                          preferred_element_type=jnp.float32)
            out_ref[...] = jnp.broadcast_to(col[None], (F, D, _BB))

        @pl.when(flag != 1)
        def _general():
            def per_field(f, c):
                idx = efft_ref[pl.ds(f, 1), :]
                oh = (viota == idx).astype(jnp.float32)
                out_ref[pl.ds(f, 1)] = jnp.dot(
                    tablet_ref[...], oh,
                    precision=lax.Precision.HIGHEST,
                    preferred_element_type=jnp.float32)[None]
                return c

            lax.fori_loop(0, F, per_field, 0)

    grid_spec = pltpu.PrefetchScalarGridSpec(
        num_scalar_prefetch=1,
        grid=(grid,),
        in_specs=[
            pl.BlockSpec((D, V), lambda i, s: (0, 0)),
            pl.BlockSpec((F, _BB), lambda i, s: (0, i)),
        ],
        out_specs=pl.BlockSpec((F, D, _BB), lambda i, s: (0, 0, i)),
    )
    return pl.pallas_call(
        body,
        grid_spec=grid_spec,
        out_shape=jax.ShapeDtypeStruct((F, D, B), jnp.float32),
        compiler_params=pltpu.CompilerParams(
            dimension_semantics=("arbitrary",)),
    )


def kernel(x, table):
    B, F = x.shape
    V, D = table.shape
    xf = x.reshape(B * F)
    eff, flags = _make_sc_analyze(B * F, V)(xf)
    efft = eff.reshape(B, F).T
    tablet = table.T
    out_fdb = _make_tc_materialize(B, F, V, D)(flags, tablet, efft)
    return out_fdb.transpose(2, 0, 1)


# speedup vs baseline: 10.0755x; 1.1952x over previous
import functools

import jax
import jax.numpy as jnp
from jax import lax
from jax.experimental import pallas as pl
from jax.experimental.pallas import tpu as pltpu
from jax.experimental.pallas import tpu_sc as plsc

_NC = 2
_NS = 16
_NW = _NC * _NS
_L = 16
_BB = 128


@functools.lru_cache(maxsize=None)
def _make_sc_analyze(N, V):
    bpw = N // _NW
    mesh = plsc.VectorSubcoreMesh(core_axis_name="c", subcore_axis_name="s")

    @functools.partial(
        pl.kernel,
        mesh=mesh,
        out_type=(
            jax.ShapeDtypeStruct((N,), jnp.int32),
            jax.ShapeDtypeStruct((_NW * _L,), jnp.int32),
        ),
        scratch_types=[
            pltpu.VMEM((bpw,), jnp.int32),
            pltpu.VMEM((_NW * _L,), jnp.int32),
        ],
        compiler_params=pltpu.CompilerParams(needs_layout_passes=False),
    )
    def body(x_hbm, eff_hbm, flags_hbm, idx_v, flag_v):
        cid = lax.axis_index("c")
        sid = lax.axis_index("s")
        wid = sid * _NC + cid
        base = pl.multiple_of(wid * bpw, 8)

        pltpu.sync_copy(x_hbm.at[pl.ds(base, bpw)], idx_v)

        def fix(j, carry):
            mn, mx = carry
            v = idx_v[pl.ds(j * _L, _L)]
            v = jnp.where(v < V, v, V - 1)
            v = jnp.where(v < 0, v, 0)
            idx_v[pl.ds(j * _L, _L)] = v
            return (jnp.minimum(mn, jnp.min(v)), jnp.maximum(mx, jnp.max(v)))

        mn, mx = lax.fori_loop(
            0, bpw // _L, fix,
            (jnp.int32(2 ** 31 - 1), jnp.int32(-(2 ** 31))))

        pltpu.sync_copy(idx_v, eff_hbm.at[pl.ds(base, bpw)])

        flag = jnp.where(mn == mx, jnp.int32(1), jnp.int32(0))
        lane = lax.broadcasted_iota(jnp.int32, (_L,), 0)
        fbase = pl.multiple_of(wid * _L, 8)
        flag_v[pl.ds(fbase, _L)] = jnp.where(lane < 8, flag, mn)
        pltpu.sync_copy(flag_v.at[pl.ds(fbase, _L)],
                        flags_hbm.at[pl.ds(fbase, _L)])

    return body


@functools.lru_cache(maxsize=None)
def _make_tc_materialize(B, F, V, D):
    grid = B // _BB

    def body(flags_s, tablet_ref, efft_ref, out_ref, col_v, cache_s):
        i = pl.program_id(0)
        flag = flags_s[i * _L]
        viota = lax.broadcasted_iota(jnp.int32, (V, _BB), 0)

        @pl.when(i == 0)
        def _init():
            cache_s[0] = jnp.int32(0)

        @pl.when(flag == 1)
        def _broadcast():
            u = flags_s[i * _L + 8]
            stale = jnp.logical_or(cache_s[0] != 1, cache_s[1] != u)

            @pl.when(stale)
            def _compute():
                oh = (viota == u).astype(jnp.float32)
                col_v[...] = jnp.dot(tablet_ref[...], oh,
                                     precision=lax.Precision.HIGHEST,
                                     preferred_element_type=jnp.float32)
                cache_s[0] = jnp.int32(1)
                cache_s[1] = u

            out_ref[...] = jnp.broadcast_to(col_v[...][None], (F, D, _BB))

        @pl.when(flag != 1)
        def _general():
            def per_field(f, c):
                idx = efft_ref[pl.ds(f, 1), :]
                oh = (viota == idx).astype(jnp.float32)
                out_ref[pl.ds(f, 1)] = jnp.dot(
                    tablet_ref[...], oh,
                    precision=lax.Precision.HIGHEST,
                    preferred_element_type=jnp.float32)[None]
                return c

            lax.fori_loop(0, F, per_field, 0)

    grid_spec = pltpu.PrefetchScalarGridSpec(
        num_scalar_prefetch=1,
        grid=(grid,),
        in_specs=[
            pl.BlockSpec((D, V), lambda i, s: (0, 0)),
            pl.BlockSpec((F, _BB), lambda i, s: (0, i)),
        ],
        out_specs=pl.BlockSpec((F, D, _BB), lambda i, s: (0, 0, i)),
        scratch_shapes=[
            pltpu.VMEM((D, _BB), jnp.float32),
            pltpu.SMEM((2,), jnp.int32),
        ],
    )
    return pl.pallas_call(
        body,
        grid_spec=grid_spec,
        out_shape=jax.ShapeDtypeStruct((F, D, B), jnp.float32),
        compiler_params=pltpu.CompilerParams(
            dimension_semantics=("arbitrary",)),
    )


def kernel(x, table):
    B, F = x.shape
    V, D = table.shape
    xf = x.reshape(B * F)
    eff, flags = _make_sc_analyze(B * F, V)(xf)
    efft = eff.reshape(B, F).T
    tablet = table.T
    out_fdb = _make_tc_materialize(B, F, V, D)(flags, tablet, efft)
    return out_fdb.transpose(2, 0, 1)


# speedup vs baseline: 10.7054x; 1.0625x over previous
import functools

import jax
import jax.numpy as jnp
from jax import lax
from jax.experimental import pallas as pl
from jax.experimental.pallas import tpu as pltpu
from jax.experimental.pallas import tpu_sc as plsc

_NC = 2
_NS = 16
_NW = _NC * _NS
_L = 16
_BB = 128


@functools.lru_cache(maxsize=None)
def _make_sc_analyze(N, V):
    bpw = N // _NW
    mesh = plsc.VectorSubcoreMesh(core_axis_name="c", subcore_axis_name="s")

    @functools.partial(
        pl.kernel,
        mesh=mesh,
        out_type=jax.ShapeDtypeStruct((_NW * _L,), jnp.int32),
        scratch_types=[
            pltpu.VMEM((bpw,), jnp.int32),
            pltpu.VMEM((_NW * _L,), jnp.int32),
        ],
        compiler_params=pltpu.CompilerParams(needs_layout_passes=False),
    )
    def body(x_hbm, flags_hbm, idx_v, flag_v):
        cid = lax.axis_index("c")
        sid = lax.axis_index("s")
        wid = sid * _NC + cid
        base = pl.multiple_of(wid * bpw, 8)

        pltpu.sync_copy(x_hbm.at[pl.ds(base, bpw)], idx_v)

        def reduce(j, carry):
            mn, mx = carry
            v = idx_v[pl.ds(j * _L, _L)]
            return (jnp.minimum(mn, jnp.min(v)), jnp.maximum(mx, jnp.max(v)))

        mn, mx = lax.fori_loop(
            0, bpw // _L, reduce,
            (jnp.int32(2 ** 31 - 1), jnp.int32(-(2 ** 31))))

        umn = jnp.minimum(mn, jnp.int32(0))
        umx = jnp.minimum(mx, jnp.int32(0))
        flag = jnp.where(umn == umx, jnp.int32(1), jnp.int32(0))

        lane = lax.broadcasted_iota(jnp.int32, (_L,), 0)
        fbase = pl.multiple_of(wid * _L, 8)
        flag_v[pl.ds(fbase, _L)] = jnp.where(lane < 8, flag, umn)
        pltpu.sync_copy(flag_v.at[pl.ds(fbase, _L)],
                        flags_hbm.at[pl.ds(fbase, _L)])

    return body


@functools.lru_cache(maxsize=None)
def _make_tc_materialize(B, F, V, D):
    grid = B // _BB
    dims = (((0,), (0,)), ((), ()))

    def body(flags_s, table_ref, xt_ref, out_ref, col_v, cache_s):
        i = pl.program_id(0)
        flag = flags_s[i * _L]
        viota = lax.broadcasted_iota(jnp.int32, (V, _BB), 0)

        @pl.when(i == 0)
        def _init():
            cache_s[0] = jnp.int32(0)

        @pl.when(flag == 1)
        def _broadcast():
            u = flags_s[i * _L + 8]
            stale = jnp.logical_or(cache_s[0] != 1, cache_s[1] != u)

            @pl.when(stale)
            def _compute():
                oh = (viota == u).astype(jnp.float32)
                col_v[...] = lax.dot_general(
                    table_ref[...], oh, dims,
                    precision=lax.Precision.HIGHEST,
                    preferred_element_type=jnp.float32)
                cache_s[0] = jnp.int32(1)
                cache_s[1] = u

            out_ref[...] = jnp.broadcast_to(col_v[...][None], (F, D, _BB))

        @pl.when(flag != 1)
        def _general():
            def per_field(f, c):
                eff = jnp.minimum(xt_ref[pl.ds(f, 1), :], 0)
                oh = (viota == eff).astype(jnp.float32)
                out_ref[pl.ds(f, 1)] = lax.dot_general(
                    table_ref[...], oh, dims,
                    precision=lax.Precision.HIGHEST,
                    preferred_element_type=jnp.float32)[None]
                return c

            lax.fori_loop(0, F, per_field, 0)

    grid_spec = pltpu.PrefetchScalarGridSpec(
        num_scalar_prefetch=1,
        grid=(grid,),
        in_specs=[
            pl.BlockSpec((V, D), lambda i, s: (0, 0)),
            pl.BlockSpec((F, _BB), lambda i, s: (0, i)),
        ],
        out_specs=pl.BlockSpec((F, D, _BB), lambda i, s: (0, 0, i)),
        scratch_shapes=[
            pltpu.VMEM((D, _BB), jnp.float32),
            pltpu.SMEM((2,), jnp.int32),
        ],
    )
    return pl.pallas_call(
        body,
        grid_spec=grid_spec,
        out_shape=jax.ShapeDtypeStruct((F, D, B), jnp.float32),
        compiler_params=pltpu.CompilerParams(
            dimension_semantics=("arbitrary",)),
    )


def kernel(x, table):
    B, F = x.shape
    V, D = table.shape
    xf = x.reshape(B * F)
    flags = _make_sc_analyze(B * F, V)(xf)
    out_fdb = _make_tc_materialize(B, F, V, D)(flags, table, x.T)
    return out_fdb.transpose(2, 0, 1)


# speedup vs baseline: 10.7561x; 1.0047x over previous
import functools

import jax
import jax.numpy as jnp
from jax import lax
from jax.experimental import pallas as pl
from jax.experimental.pallas import tpu as pltpu
from jax.experimental.pallas import tpu_sc as plsc

_NC = 2
_NS = 16
_NW = _NC * _NS
_L = 16
_BB = 256


@functools.lru_cache(maxsize=None)
def _make_sc_analyze(N, V):
    bpw = N // _NW
    mesh = plsc.VectorSubcoreMesh(core_axis_name="c", subcore_axis_name="s")

    @functools.partial(
        pl.kernel,
        mesh=mesh,
        out_type=jax.ShapeDtypeStruct((_NW * _L,), jnp.int32),
        scratch_types=[
            pltpu.VMEM((bpw,), jnp.int32),
            pltpu.VMEM((_NW * _L,), jnp.int32),
        ],
        compiler_params=pltpu.CompilerParams(needs_layout_passes=False),
    )
    def body(x_hbm, flags_hbm, idx_v, flag_v):
        cid = lax.axis_index("c")
        sid = lax.axis_index("s")
        wid = sid * _NC + cid
        base = pl.multiple_of(wid * bpw, 8)

        pltpu.sync_copy(x_hbm.at[pl.ds(base, bpw)], idx_v)

        def reduce(j, carry):
            mn, mx = carry
            v = idx_v[pl.ds(j * _L, _L)]
            return (jnp.minimum(mn, jnp.min(v)), jnp.maximum(mx, jnp.max(v)))

        mn, mx = lax.fori_loop(
            0, bpw // _L, reduce,
            (jnp.int32(2 ** 31 - 1), jnp.int32(-(2 ** 31))))

        umn = jnp.minimum(mn, jnp.int32(0))
        umx = jnp.minimum(mx, jnp.int32(0))
        flag = jnp.where(umn == umx, jnp.int32(1), jnp.int32(0))

        lane = lax.broadcasted_iota(jnp.int32, (_L,), 0)
        fbase = pl.multiple_of(wid * _L, 8)
        flag_v[pl.ds(fbase, _L)] = jnp.where(lane < 8, flag, umn)
        pltpu.sync_copy(flag_v.at[pl.ds(fbase, _L)],
                        flags_hbm.at[pl.ds(fbase, _L)])

    return body


@functools.lru_cache(maxsize=None)
def _make_tc_materialize(B, F, V, D):
    grid = B // _BB
    dims = (((0,), (0,)), ((), ()))

    def body(flags_s, table_ref, xt_ref, out_ref, col_v, cache_s):
        i = pl.program_id(0)
        viota = lax.broadcasted_iota(jnp.int32, (V, _BB), 0)

        u = flags_s[8]

        def red(w, gf):
            fw = flags_s[w * _L]
            uw = flags_s[w * _L + 8]
            return gf & jnp.where((fw == 1) & (uw == u), 1, 0)

        flag = lax.fori_loop(0, _NW, red, jnp.int32(1))

        @pl.when(i == 0)
        def _init():
            cache_s[0] = jnp.int32(0)

        @pl.when(flag == 1)
        def _broadcast():
            stale = jnp.logical_or(cache_s[0] != 1, cache_s[1] != u)

            @pl.when(stale)
            def _compute():
                oh = (viota == u).astype(jnp.float32)
                col_v[...] = lax.dot_general(
                    table_ref[...], oh, dims,
                    precision=lax.Precision.HIGHEST,
                    preferred_element_type=jnp.float32)
                cache_s[0] = jnp.int32(1)
                cache_s[1] = u

            out_ref[...] = jnp.broadcast_to(col_v[...][None], (F, D, _BB))

        @pl.when(flag != 1)
        def _general():
            def per_field(f, c):
                eff = jnp.minimum(xt_ref[pl.ds(f, 1), :], 0)
                oh = (viota == eff).astype(jnp.float32)
                out_ref[pl.ds(f, 1)] = lax.dot_general(
                    table_ref[...], oh, dims,
                    precision=lax.Precision.HIGHEST,
                    preferred_element_type=jnp.float32)[None]
                return c

            lax.fori_loop(0, F, per_field, 0)

    grid_spec = pltpu.PrefetchScalarGridSpec(
        num_scalar_prefetch=1,
        grid=(grid,),
        in_specs=[
            pl.BlockSpec((V, D), lambda i, s: (0, 0)),
            pl.BlockSpec((F, _BB), lambda i, s: (0, i)),
        ],
        out_specs=pl.BlockSpec((F, D, _BB), lambda i, s: (0, 0, i)),
        scratch_shapes=[
            pltpu.VMEM((D, _BB), jnp.float32),
            pltpu.SMEM((2,), jnp.int32),
        ],
    )
    return pl.pallas_call(
        body,
        grid_spec=grid_spec,
        out_shape=jax.ShapeDtypeStruct((F, D, B), jnp.float32),
        compiler_params=pltpu.CompilerParams(
            dimension_semantics=("arbitrary",)),
    )


def kernel(x, table):
    B, F = x.shape
    V, D = table.shape
    xt = x.T
    xtf = xt.reshape(B * F)
    flags = _make_sc_analyze(B * F, V)(xtf)
    out_fdb = _make_tc_materialize(B, F, V, D)(flags, table, xt)
    return out_fdb.transpose(2, 0, 1)
